# baseline (speedup 1.0000x reference)
import functools

import jax
import jax.numpy as jnp
from jax import lax
from jax.experimental import pallas as pl
from jax.experimental.pallas import tpu as pltpu
from jax.experimental.pallas import tpu_sc as plsc

N = 10000
E = 320000
E_PAD = 327680
IN = 128
HID = 128
H = 8
C = 64

NC = 2
NS = 16
NPT_STEP = 624
NPT_SIZE = 640
BN = 1000
KA = 1024
KB = 1024

_SC_PARAMS = pltpu.CompilerParams(use_tc_tiling_on_sc=False,
                                  needs_layout_passes=False)


def _mesh():
    return plsc.VectorSubcoreMesh(core_axis_name="c", subcore_axis_name="s")


def _matmul_layer(h, W, ALc, ARc, nslices):
    n, K = h.shape
    M = W.shape[1]

    def body(h_ref, w_ref, al_ref, ar_ref, feat_ref, el_ref, er_ref):
        acc = jnp.dot(h_ref[...], w_ref[...], preferred_element_type=jnp.float32)
        for t in range(nslices):
            feat_ref[t] = acc[:, t * 64:(t + 1) * 64]
        el_ref[...] = jnp.dot(acc, al_ref[...], preferred_element_type=jnp.float32)
        er_ref[...] = jnp.dot(acc, ar_ref[...], preferred_element_type=jnp.float32)

    return pl.pallas_call(
        body,
        grid=(n // BN,),
        in_specs=[
            pl.BlockSpec((BN, K), lambda i: (i, 0)),
            pl.BlockSpec((K, M), lambda i: (0, 0)),
            pl.BlockSpec((M, 8), lambda i: (0, 0)),
            pl.BlockSpec((M, 8), lambda i: (0, 0)),
        ],
        out_specs=[
            pl.BlockSpec((nslices, BN, 64), lambda i: (0, i, 0)),
            pl.BlockSpec((BN, 8), lambda i: (i, 0)),
            pl.BlockSpec((BN, 8), lambda i: (i, 0)),
        ],
        out_shape=[
            jax.ShapeDtypeStruct((nslices, n, 64), jnp.float32),
            jax.ShapeDtypeStruct((n, 8), jnp.float32),
            jax.ShapeDtypeStruct((n, 8), jnp.float32),
        ],
    )(h, W, ALc, ARc)


def _attn_den(el, erp, src2d, dst2d, z8):
    mesh = _mesh()
    ept = E_PAD // (NC * NS)
    nb = ept // KA
    nch = KA // 128

    @functools.partial(
        pl.kernel,
        out_type=[
            jax.ShapeDtypeStruct((H, E_PAD), jnp.float32),
            jax.ShapeDtypeStruct((NC, N + 8, 8), jnp.float32),
        ],
        mesh=mesh,
        compiler_params=_SC_PARAMS,
        scratch_types=[
            pltpu.VMEM((8, 128), jnp.int32),
            pltpu.VMEM((8, 128), jnp.int32),
            pltpu.VMEM((KA, 8), jnp.float32),
            pltpu.VMEM((KA, 8), jnp.float32),
            pltpu.VMEM((KA, 8), jnp.float32),
            pltpu.VMEM((8, KA), jnp.float32),
            pltpu.VMEM_SHARED((N + 8, 8), jnp.float32),
            pltpu.SemaphoreType.DMA,
            pltpu.SemaphoreType.DMA,
        ],
    )
    def k(el_hbm, er_hbm, src_hbm, dst_hbm, z_hbm, ex_hbm, den_hbm,
          srcv, dstv, elv, erv, exv, exhm, den_sh, sem1, sem2):
        c = lax.axis_index("c")
        s = lax.axis_index("s")
        wid = s * NC + c
        cbase = wid * (ept // 128)
        pltpu.sync_copy(z_hbm.at[pl.ds(s * NPT_STEP, NPT_SIZE)],
                        den_sh.at[pl.ds(s * NPT_STEP, NPT_SIZE)])
        plsc.subcore_barrier()
        iot = lax.iota(jnp.int32, 16)
        rows0 = iot // 8
        cols = iot % 8

        def batch(b, carry):
            cb = cbase + b * nch
            pltpu.sync_copy(src_hbm.at[pl.ds(cb, nch)], srcv)
            pltpu.sync_copy(dst_hbm.at[pl.ds(cb, nch)], dstv)
            cps = []
            for ch in range(nch):
                cps.append(pltpu.async_copy(
                    el_hbm.at[srcv.at[ch]], elv.at[pl.ds(ch * 128, 128)], sem1))
                cps.append(pltpu.async_copy(
                    er_hbm.at[dstv.at[ch]], erv.at[pl.ds(ch * 128, 128)], sem2))
            for cp in cps:
                cp.wait()

            def grp(g, c2):
                rows = rows0 + 2 * g
                e = plsc.load_gather(elv, [rows, cols]) + plsc.load_gather(erv, [rows, cols])
                e = jnp.where(e > 0.0, e, 0.2 * e)
                ex16 = jnp.exp(e)
                plsc.store_scatter(exv, [rows, cols], ex16)
                plsc.store_scatter(exhm, [cols, rows], ex16)
                return c2

            lax.fori_loop(0, KA // 2, grp, 0)
            for hq in range(H):
                pltpu.sync_copy(exhm.at[hq], ex_hbm.at[hq, pl.ds(cb * 128, KA)])
            for ch in range(nch):
                pltpu.sync_copy(exv.at[pl.ds(ch * 128, 128)],
                                den_sh.at[dstv.at[ch]], add=True)
            return carry

        lax.fori_loop(0, nb, batch, 0)
        plsc.subcore_barrier()
        pltpu.sync_copy(den_sh.at[pl.ds(s * NPT_STEP, NPT_SIZE)],
                        den_hbm.at[c, pl.ds(s * NPT_STEP, NPT_SIZE)])

    return k(el, erp, src2d, dst2d, z8)


KB2 = 640


def _msg(feat_table, ex, src2d, dst2d, z64, nheads):
    mesh = _mesh()
    if nheads == 8:
        ept = E_PAD // NS
        out_shape = jax.ShapeDtypeStruct((2 * H, N, C), jnp.float32)
        ntab = 8
    else:
        ept = E_PAD // (NC * NS)
        out_shape = jax.ShapeDtypeStruct((NC, N, C), jnp.float32)
        ntab = 1
    nb = ept // KB2
    nouter = nb // 2
    nch = KB2 // 128

    idx_t = pltpu.VMEM((nch, 128), jnp.int32)

    @functools.partial(
        pl.kernel,
        out_type=out_shape,
        mesh=mesh,
        compiler_params=_SC_PARAMS,
        scratch_types=[
            idx_t, idx_t,
            idx_t, idx_t,
            idx_t, idx_t,
            pltpu.VMEM((KB2,), jnp.float32),
            pltpu.VMEM((KB2,), jnp.float32),
            pltpu.VMEM((KB2, C), jnp.float32),
            pltpu.VMEM((KB2, C), jnp.float32),
            pltpu.VMEM_SHARED((N + 8, C), jnp.float32),
            pltpu.SemaphoreType.DMA,
            pltpu.SemaphoreType.DMA,
            pltpu.SemaphoreType.DMA,
            pltpu.SemaphoreType.DMA,
        ],
    )
    def k(feat_hbm, ex_hbm, src_hbm, dst_hbm, z_hbm, out_hbm,
          srcv0, srcv1, srcw0, srcw1, dstv0, dstv1, exv0, exv1,
          featv0, featv1, rst_sh, gs0, gs1, ss0, ss1):
        c = lax.axis_index("c")
        s = lax.axis_index("s")
        wid = s * NC + c
        srcvs = (srcv0, srcv1)
        srcws = (srcw0, srcw1)
        dstvs = (dstv0, dstv1)
        exvs = (exv0, exv1)
        featvs = (featv0, featv1)
        gss = (gs0, gs1)
        sss = (ss0, ss1)
        iot = lax.iota(jnp.int32, 16)
        for l in range(ntab):
            if nheads == 8:
                tab = c * 8 + l
                hh = c * 4 + l // 2
                cbase = s * (ept // 128)
            else:
                tab = 0
                hh = 0
                cbase = wid * (ept // 128)
            pltpu.sync_copy(z_hbm.at[pl.ds(s * NPT_STEP, NPT_SIZE)],
                            rst_sh.at[pl.ds(s * NPT_STEP, NPT_SIZE)])
            plsc.subcore_barrier()

            def stage_fire(i, q):
                cb = cbase + i * nch
                pltpu.sync_copy(src_hbm.at[pl.ds(cb, nch)], srcvs[q])
                pltpu.sync_copy(dst_hbm.at[pl.ds(cb, nch)], dstvs[q])
                pltpu.async_copy(ex_hbm.at[hh, pl.ds(cb * 128, KB2)], exvs[q], gss[q])
                if nheads == 8:
                    tN = tab * N
                    for ch in range(nch):
                        for qq in range(8):
                            srcws[q][ch, pl.ds(qq * 16, 16)] = (
                                srcvs[q][ch, pl.ds(qq * 16, 16)] + tN)
                    idxr = srcws[q]
                else:
                    idxr = srcvs[q]
                for ch in range(nch):
                    pltpu.async_copy(feat_hbm.at[idxr.at[ch]],
                                     featvs[q].at[pl.ds(ch * 128, 128)], gss[q])

            def wait_gather(q):
                pltpu.make_async_copy(feat_hbm.at[pl.ds(0, KB2)], featvs[q],
                                      gss[q]).wait()
                pltpu.make_async_copy(ex_hbm.at[0, pl.ds(0, KB2)], exvs[q],
                                      gss[q]).wait()

            def compute(q):
                exv = exvs[q]
                featv = featvs[q]

                def grp(g, c2):
                    alpha = exv[pl.ds(g * 16, 16)]
                    for j in range(16):
                        r = g * 16 + j
                        bc = jnp.take(alpha, jnp.full((16,), j, jnp.int32))
                        for kk in range(C // 16):
                            featv[r, pl.ds(kk * 16, 16)] = (
                                featv[r, pl.ds(kk * 16, 16)] * bc)
                    return c2

                lax.fori_loop(0, KB2 // 16, grp, 0)

            def fire_scatter(q):
                for ch in range(nch):
                    pltpu.async_copy(featvs[q].at[pl.ds(ch * 128, 128)],
                                     rst_sh.at[dstvs[q].at[ch]], sss[q],
                                     add=True)

            def wait_scatter(q):
                pltpu.make_async_copy(feat_hbm.at[pl.ds(0, KB2)], featvs[q],
                                      sss[q]).wait()

            stage_fire(0, 0)
            stage_fire(1, 1)

            def outer(g, carry):
                wait_gather(0)
                compute(0)
                fire_scatter(0)
                wait_gather(1)
                compute(1)
                fire_scatter(1)

                @pl.when(g < nouter - 1)
                def _():
                    wait_scatter(0)
                    stage_fire(2 * g + 2, 0)
                    wait_scatter(1)
                    stage_fire(2 * g + 3, 1)

                return carry

            lax.fori_loop(0, nouter, outer, 0)
            wait_scatter(0)
            wait_scatter(1)
            plsc.subcore_barrier()
            if nheads == 8:
                pltpu.sync_copy(rst_sh.at[pl.ds(s * NPT_STEP, NPT_SIZE)],
                                out_hbm.at[tab, pl.ds(s * NPT_STEP, NPT_SIZE)])
            else:
                pltpu.sync_copy(rst_sh.at[pl.ds(s * NPT_STEP, NPT_SIZE)],
                                out_hbm.at[c, pl.ds(s * NPT_STEP, NPT_SIZE)])
            plsc.subcore_barrier()

    return k(feat_table, ex, src2d, dst2d, z64)


def _epilogue(rst, denA, denB, res_flat, bias2d, act):
    n = rst.shape[1]
    have_res = res_flat is not None

    def body(*refs):
        if have_res:
            rst_ref, dA_ref, dB_ref, res_ref, b_ref, o_ref = refs
        else:
            rst_ref, dA_ref, dB_ref, b_ref, o_ref = refs
        den = dA_ref[...] + dB_ref[...] + 1e-9
        for t in range(16):
            hd = t // 2
            lo = t * 64
            v = rst_ref[t] / den[:, hd:hd + 1]
            v = v + b_ref[0:1, lo:lo + 64]
            if have_res:
                v = v + res_ref[:, lo:lo + 64]
            if act:
                v = jnp.maximum(v, 0.0)
            o_ref[:, lo:lo + 64] = v

    in_specs = [
        pl.BlockSpec((16, BN, 64), lambda i: (0, i, 0)),
        pl.BlockSpec((BN, 8), lambda i: (i, 0)),
        pl.BlockSpec((BN, 8), lambda i: (i, 0)),
    ]
    args = [rst, denA, denB]
    if have_res:
        in_specs.append(pl.BlockSpec((BN, H * HID), lambda i: (i, 0)))
        args.append(res_flat)
    in_specs.append(pl.BlockSpec((1, H * HID), lambda i: (0, 0)))
    args.append(bias2d)
    return pl.pallas_call(
        body,
        grid=(n // BN,),
        in_specs=in_specs,
        out_specs=pl.BlockSpec((BN, H * HID), lambda i: (i, 0)),
        out_shape=jax.ShapeDtypeStruct((n, H * HID), jnp.float32),
    )(*args)


def _epilogue2(rstA, rstB, denA, denB, resv, bias2d):
    n = rstA.shape[0]

    def body(rA_ref, rB_ref, dA_ref, dB_ref, res_ref, b_ref, o_ref):
        den = dA_ref[:, 0:1] + dB_ref[:, 0:1] + 1e-9
        o_ref[...] = (rA_ref[...] + rB_ref[...]) / den + res_ref[...] + b_ref[0:1, :]

    return pl.pallas_call(
        body,
        grid=(n // BN,),
        in_specs=[
            pl.BlockSpec((BN, C), lambda i: (i, 0)),
            pl.BlockSpec((BN, C), lambda i: (i, 0)),
            pl.BlockSpec((BN, 8), lambda i: (i, 0)),
            pl.BlockSpec((BN, 8), lambda i: (i, 0)),
            pl.BlockSpec((BN, C), lambda i: (i, 0)),
            pl.BlockSpec((1, C), lambda i: (0, 0)),
        ],
        out_specs=pl.BlockSpec((BN, C), lambda i: (i, 0)),
        out_shape=jax.ShapeDtypeStruct((n, C), jnp.float32),
    )(rstA, rstB, denA, denB, resv, bias2d)


def _blockdiag(al):
    h, d = al.shape
    return (jnp.eye(h, dtype=al.dtype)[:, None, :] * al[:, :, None]).reshape(h * d, h)


def _pad_er(er):
    return jnp.concatenate([er, jnp.zeros((8, 8), er.dtype)], axis=0)


def kernel(inputs, edge_index, W0, al0, ar0, b0, W1, al1, ar1, b1, W2, al2, ar2, b2, res2):
    src = edge_index[0].astype(jnp.int32)
    dst = edge_index[1].astype(jnp.int32)
    npad = E_PAD - E
    src2d = jnp.concatenate([src, jnp.zeros((npad,), jnp.int32)]).reshape(E_PAD // 128, 128)
    dst2d = jnp.concatenate([dst, jnp.full((npad,), N, jnp.int32)]).reshape(E_PAD // 128, 128)
    h0 = inputs[0]
    z8 = jnp.zeros((N, 8), jnp.float32)
    z64 = jnp.zeros((N, C), jnp.float32)

    feat0, el0, er0 = _matmul_layer(h0, W0, _blockdiag(al0), _blockdiag(ar0), 16)
    ex0, den0 = _attn_den(el0, _pad_er(er0), src2d, dst2d, z8)
    rst0 = _msg(feat0.reshape(16 * N, C), ex0, src2d, dst2d, z64, H)
    h1 = _epilogue(rst0, den0[0, :N], den0[1, :N], None, b0.reshape(1, H * HID), act=True)

    feat1, el1, er1 = _matmul_layer(h1, W1, _blockdiag(al1), _blockdiag(ar1), 16)
    ex1, den1 = _attn_den(el1, _pad_er(er1), src2d, dst2d, z8)
    rst1 = _msg(feat1.reshape(16 * N, C), ex1, src2d, dst2d, z64, H)
    h2 = _epilogue(rst1, den1[0, :N], den1[1, :N], h1, b1.reshape(1, H * HID), act=True)

    Wc = jnp.concatenate([W2, res2], axis=1)
    ALc2 = jnp.zeros((2 * C, 8), jnp.float32).at[:C, 0].set(al2[0])
    ARc2 = jnp.zeros((2 * C, 8), jnp.float32).at[:C, 0].set(ar2[0])
    featc, el2, er2 = _matmul_layer(h2, Wc, ALc2, ARc2, 2)
    ex2, den2 = _attn_den(el2, _pad_er(er2), src2d, dst2d, z8)
    rst2 = _msg(featc[0], ex2, src2d, dst2d, z64, 1)
    logits = _epilogue2(rst2[0], rst2[1], den2[0, :N], den2[1, :N],
                        featc[1], b2.reshape(1, C))
    return logits[:N - 1]


# speedup vs baseline: 1.1359x; 1.1359x over previous
import functools

import jax
import jax.numpy as jnp
from jax import lax
from jax.experimental import pallas as pl
from jax.experimental.pallas import tpu as pltpu
from jax.experimental.pallas import tpu_sc as plsc

N = 10000
E = 320000
E_PAD = 327680
IN = 128
HID = 128
H = 8
C = 64

NC = 2
NS = 16
NPT_STEP = 624
NPT_SIZE = 640
BN = 1000
KA = 1024
KB = 1024

_SC_PARAMS = pltpu.CompilerParams(use_tc_tiling_on_sc=False,
                                  needs_layout_passes=False)


def _mesh():
    return plsc.VectorSubcoreMesh(core_axis_name="c", subcore_axis_name="s")


def _matmul_layer(h, W, ALc, ARc, nslices):
    n, K = h.shape
    M = W.shape[1]

    def body(h_ref, w_ref, al_ref, ar_ref, feat_ref, el_ref, er_ref):
        acc = jnp.dot(h_ref[...], w_ref[...], preferred_element_type=jnp.float32)
        for t in range(nslices):
            feat_ref[t] = acc[:, t * 64:(t + 1) * 64]
        el_ref[...] = jnp.dot(acc, al_ref[...], preferred_element_type=jnp.float32)
        er_ref[...] = jnp.dot(acc, ar_ref[...], preferred_element_type=jnp.float32)

    return pl.pallas_call(
        body,
        grid=(n // BN,),
        in_specs=[
            pl.BlockSpec((BN, K), lambda i: (i, 0)),
            pl.BlockSpec((K, M), lambda i: (0, 0)),
            pl.BlockSpec((M, 8), lambda i: (0, 0)),
            pl.BlockSpec((M, 8), lambda i: (0, 0)),
        ],
        out_specs=[
            pl.BlockSpec((nslices, BN, 64), lambda i: (0, i, 0)),
            pl.BlockSpec((BN, 8), lambda i: (i, 0)),
            pl.BlockSpec((BN, 8), lambda i: (i, 0)),
        ],
        out_shape=[
            jax.ShapeDtypeStruct((nslices, n, 64), jnp.float32),
            jax.ShapeDtypeStruct((n, 8), jnp.float32),
            jax.ShapeDtypeStruct((n, 8), jnp.float32),
        ],
    )(h, W, ALc, ARc)


def _attn_den(el, erp, src2d, dst2d, z8):
    mesh = _mesh()
    ept = E_PAD // (NC * NS)
    nb = ept // KA
    nch = KA // 128

    @functools.partial(
        pl.kernel,
        out_type=[
            jax.ShapeDtypeStruct((H, E_PAD), jnp.float32),
            jax.ShapeDtypeStruct((NC, N + 8, 8), jnp.float32),
        ],
        mesh=mesh,
        compiler_params=_SC_PARAMS,
        scratch_types=[
            pltpu.VMEM((8, 128), jnp.int32),
            pltpu.VMEM((8, 128), jnp.int32),
            pltpu.VMEM((KA, 8), jnp.float32),
            pltpu.VMEM((KA, 8), jnp.float32),
            pltpu.VMEM((KA, 8), jnp.float32),
            pltpu.VMEM((8, KA), jnp.float32),
            pltpu.VMEM_SHARED((N + 8, 8), jnp.float32),
            pltpu.SemaphoreType.DMA,
            pltpu.SemaphoreType.DMA,
        ],
    )
    def k(el_hbm, er_hbm, src_hbm, dst_hbm, z_hbm, ex_hbm, den_hbm,
          srcv, dstv, elv, erv, exv, exhm, den_sh, sem1, sem2):
        c = lax.axis_index("c")
        s = lax.axis_index("s")
        wid = s * NC + c
        cbase = wid * (ept // 128)
        pltpu.sync_copy(z_hbm.at[pl.ds(s * NPT_STEP, NPT_SIZE)],
                        den_sh.at[pl.ds(s * NPT_STEP, NPT_SIZE)])
        plsc.subcore_barrier()
        iot = lax.iota(jnp.int32, 16)
        rows0 = iot // 8
        cols = iot % 8

        def batch(b, carry):
            cb = cbase + b * nch
            pltpu.sync_copy(src_hbm.at[pl.ds(cb, nch)], srcv)
            pltpu.sync_copy(dst_hbm.at[pl.ds(cb, nch)], dstv)
            cps = []
            for ch in range(nch):
                cps.append(pltpu.async_copy(
                    el_hbm.at[srcv.at[ch]], elv.at[pl.ds(ch * 128, 128)], sem1))
                cps.append(pltpu.async_copy(
                    er_hbm.at[dstv.at[ch]], erv.at[pl.ds(ch * 128, 128)], sem2))
            for cp in cps:
                cp.wait()

            def grp(g, c2):
                rows = rows0 + 2 * g
                e = plsc.load_gather(elv, [rows, cols]) + plsc.load_gather(erv, [rows, cols])
                e = jnp.where(e > 0.0, e, 0.2 * e)
                ex16 = jnp.exp(e)
                plsc.store_scatter(exv, [rows, cols], ex16)
                plsc.store_scatter(exhm, [cols, rows], ex16)
                return c2

            lax.fori_loop(0, KA // 2, grp, 0)
            for hq in range(H):
                pltpu.sync_copy(exhm.at[hq], ex_hbm.at[hq, pl.ds(cb * 128, KA)])
            for ch in range(nch):
                pltpu.sync_copy(exv.at[pl.ds(ch * 128, 128)],
                                den_sh.at[dstv.at[ch]], add=True)
            return carry

        lax.fori_loop(0, nb, batch, 0)
        plsc.subcore_barrier()
        pltpu.sync_copy(den_sh.at[pl.ds(s * NPT_STEP, NPT_SIZE)],
                        den_hbm.at[c, pl.ds(s * NPT_STEP, NPT_SIZE)])

    return k(el, erp, src2d, dst2d, z8)


KB2 = 640


def _msg(feat_table, ex, src2d, dst2d, z64, nheads):
    mesh = _mesh()
    if nheads == 8:
        ept = E_PAD // NS
        out_shape = jax.ShapeDtypeStruct((2 * H, N, C), jnp.float32)
        ntab = 8
    else:
        ept = E_PAD // (NC * NS)
        out_shape = jax.ShapeDtypeStruct((NC, N, C), jnp.float32)
        ntab = 1
    nb = ept // KB2
    nouter = nb // 2
    nch = KB2 // 128

    idx_t = pltpu.VMEM((nch, 128), jnp.int32)

    @functools.partial(
        pl.kernel,
        out_type=out_shape,
        mesh=mesh,
        compiler_params=_SC_PARAMS,
        scratch_types=[
            idx_t, idx_t,
            idx_t, idx_t,
            idx_t, idx_t,
            pltpu.VMEM((KB2,), jnp.float32),
            pltpu.VMEM((KB2,), jnp.float32),
            pltpu.VMEM((KB2, C), jnp.float32),
            pltpu.VMEM((KB2, C), jnp.float32),
            pltpu.VMEM_SHARED((N + 8, C), jnp.float32),
            pltpu.SemaphoreType.DMA,
            pltpu.SemaphoreType.DMA,
            pltpu.SemaphoreType.DMA,
            pltpu.SemaphoreType.DMA,
        ],
    )
    def k(feat_hbm, ex_hbm, src_hbm, dst_hbm, z_hbm, out_hbm,
          srcv0, srcv1, srcw0, srcw1, dstv0, dstv1, exv0, exv1,
          featv0, featv1, rst_sh, gs0, gs1, ss0, ss1):
        c = lax.axis_index("c")
        s = lax.axis_index("s")
        wid = s * NC + c
        srcvs = (srcv0, srcv1)
        srcws = (srcw0, srcw1)
        dstvs = (dstv0, dstv1)
        exvs = (exv0, exv1)
        featvs = (featv0, featv1)
        gss = (gs0, gs1)
        sss = (ss0, ss1)
        iot = lax.iota(jnp.int32, 16)
        for l in range(ntab):
            if nheads == 8:
                tab = c * 8 + l
                hh = c * 4 + l // 2
                cbase = s * (ept // 128)
            else:
                tab = 0
                hh = 0
                cbase = wid * (ept // 128)
            pltpu.sync_copy(z_hbm.at[pl.ds(s * NPT_STEP, NPT_SIZE)],
                            rst_sh.at[pl.ds(s * NPT_STEP, NPT_SIZE)])
            plsc.subcore_barrier()

            def stage_fire(i, q):
                cb = cbase + i * nch
                pltpu.sync_copy(src_hbm.at[pl.ds(cb, nch)], srcvs[q])
                pltpu.sync_copy(dst_hbm.at[pl.ds(cb, nch)], dstvs[q])
                pltpu.async_copy(ex_hbm.at[hh, pl.ds(cb * 128, KB2)], exvs[q], gss[q])
                if nheads == 8:
                    tN = tab * N
                    for ch in range(nch):
                        for qq in range(8):
                            srcws[q][ch, pl.ds(qq * 16, 16)] = (
                                srcvs[q][ch, pl.ds(qq * 16, 16)] + tN)
                    idxr = srcws[q]
                else:
                    idxr = srcvs[q]
                for ch in range(nch):
                    pltpu.async_copy(feat_hbm.at[idxr.at[ch]],
                                     featvs[q].at[pl.ds(ch * 128, 128)], gss[q])

            def wait_gather(q):
                pltpu.make_async_copy(feat_hbm.at[pl.ds(0, KB2)], featvs[q],
                                      gss[q]).wait()
                pltpu.make_async_copy(ex_hbm.at[0, pl.ds(0, KB2)], exvs[q],
                                      gss[q]).wait()

            def compute(q):
                exv = exvs[q]
                featv = featvs[q]

                @functools.partial(plsc.parallel_loop, 0, KB2 // 16, unroll=2)
                def grp(g):
                    alpha = exv[pl.ds(g * 16, 16)]
                    for j in range(16):
                        r = g * 16 + j
                        bc = jnp.take(alpha, jnp.full((16,), j, jnp.int32))
                        for kk in range(C // 16):
                            featv[r, pl.ds(kk * 16, 16)] = (
                                featv[r, pl.ds(kk * 16, 16)] * bc)

            def fire_scatter(q):
                for ch in range(nch):
                    pltpu.async_copy(featvs[q].at[pl.ds(ch * 128, 128)],
                                     rst_sh.at[dstvs[q].at[ch]], sss[q],
                                     add=True)

            def wait_scatter(q):
                pltpu.make_async_copy(feat_hbm.at[pl.ds(0, KB2)], featvs[q],
                                      sss[q]).wait()

            stage_fire(0, 0)
            stage_fire(1, 1)

            def outer(g, carry):
                wait_gather(0)
                compute(0)
                fire_scatter(0)
                wait_gather(1)
                compute(1)
                fire_scatter(1)

                @pl.when(g < nouter - 1)
                def _():
                    wait_scatter(0)
                    stage_fire(2 * g + 2, 0)
                    wait_scatter(1)
                    stage_fire(2 * g + 3, 1)

                return carry

            lax.fori_loop(0, nouter, outer, 0)
            wait_scatter(0)
            wait_scatter(1)
            plsc.subcore_barrier()
            if nheads == 8:
                pltpu.sync_copy(rst_sh.at[pl.ds(s * NPT_STEP, NPT_SIZE)],
                                out_hbm.at[tab, pl.ds(s * NPT_STEP, NPT_SIZE)])
            else:
                pltpu.sync_copy(rst_sh.at[pl.ds(s * NPT_STEP, NPT_SIZE)],
                                out_hbm.at[c, pl.ds(s * NPT_STEP, NPT_SIZE)])
            plsc.subcore_barrier()

    return k(feat_table, ex, src2d, dst2d, z64)


def _epilogue(rst, denA, denB, res_flat, bias2d, act):
    n = rst.shape[1]
    have_res = res_flat is not None

    def body(*refs):
        if have_res:
            rst_ref, dA_ref, dB_ref, res_ref, b_ref, o_ref = refs
        else:
            rst_ref, dA_ref, dB_ref, b_ref, o_ref = refs
        den = dA_ref[...] + dB_ref[...] + 1e-9
        for t in range(16):
            hd = t // 2
            lo = t * 64
            v = rst_ref[t] / den[:, hd:hd + 1]
            v = v + b_ref[0:1, lo:lo + 64]
            if have_res:
                v = v + res_ref[:, lo:lo + 64]
            if act:
                v = jnp.maximum(v, 0.0)
            o_ref[:, lo:lo + 64] = v

    in_specs = [
        pl.BlockSpec((16, BN, 64), lambda i: (0, i, 0)),
        pl.BlockSpec((BN, 8), lambda i: (i, 0)),
        pl.BlockSpec((BN, 8), lambda i: (i, 0)),
    ]
    args = [rst, denA, denB]
    if have_res:
        in_specs.append(pl.BlockSpec((BN, H * HID), lambda i: (i, 0)))
        args.append(res_flat)
    in_specs.append(pl.BlockSpec((1, H * HID), lambda i: (0, 0)))
    args.append(bias2d)
    return pl.pallas_call(
        body,
        grid=(n // BN,),
        in_specs=in_specs,
        out_specs=pl.BlockSpec((BN, H * HID), lambda i: (i, 0)),
        out_shape=jax.ShapeDtypeStruct((n, H * HID), jnp.float32),
    )(*args)


def _epilogue2(rstA, rstB, denA, denB, resv, bias2d):
    n = rstA.shape[0]

    def body(rA_ref, rB_ref, dA_ref, dB_ref, res_ref, b_ref, o_ref):
        den = dA_ref[:, 0:1] + dB_ref[:, 0:1] + 1e-9
        o_ref[...] = (rA_ref[...] + rB_ref[...]) / den + res_ref[...] + b_ref[0:1, :]

    return pl.pallas_call(
        body,
        grid=(n // BN,),
        in_specs=[
            pl.BlockSpec((BN, C), lambda i: (i, 0)),
            pl.BlockSpec((BN, C), lambda i: (i, 0)),
            pl.BlockSpec((BN, 8), lambda i: (i, 0)),
            pl.BlockSpec((BN, 8), lambda i: (i, 0)),
            pl.BlockSpec((BN, C), lambda i: (i, 0)),
            pl.BlockSpec((1, C), lambda i: (0, 0)),
        ],
        out_specs=pl.BlockSpec((BN, C), lambda i: (i, 0)),
        out_shape=jax.ShapeDtypeStruct((n, C), jnp.float32),
    )(rstA, rstB, denA, denB, resv, bias2d)


def _blockdiag(al):
    h, d = al.shape
    return (jnp.eye(h, dtype=al.dtype)[:, None, :] * al[:, :, None]).reshape(h * d, h)


def _pad_er(er):
    return jnp.concatenate([er, jnp.zeros((8, 8), er.dtype)], axis=0)


def kernel(inputs, edge_index, W0, al0, ar0, b0, W1, al1, ar1, b1, W2, al2, ar2, b2, res2):
    src = edge_index[0].astype(jnp.int32)
    dst = edge_index[1].astype(jnp.int32)
    npad = E_PAD - E
    src2d = jnp.concatenate([src, jnp.zeros((npad,), jnp.int32)]).reshape(E_PAD // 128, 128)
    dst2d = jnp.concatenate([dst, jnp.full((npad,), N, jnp.int32)]).reshape(E_PAD // 128, 128)
    h0 = inputs[0]
    z8 = jnp.zeros((N, 8), jnp.float32)
    z64 = jnp.zeros((N, C), jnp.float32)

    feat0, el0, er0 = _matmul_layer(h0, W0, _blockdiag(al0), _blockdiag(ar0), 16)
    ex0, den0 = _attn_den(el0, _pad_er(er0), src2d, dst2d, z8)
    rst0 = _msg(feat0.reshape(16 * N, C), ex0, src2d, dst2d, z64, H)
    h1 = _epilogue(rst0, den0[0, :N], den0[1, :N], None, b0.reshape(1, H * HID), act=True)

    feat1, el1, er1 = _matmul_layer(h1, W1, _blockdiag(al1), _blockdiag(ar1), 16)
    ex1, den1 = _attn_den(el1, _pad_er(er1), src2d, dst2d, z8)
    rst1 = _msg(feat1.reshape(16 * N, C), ex1, src2d, dst2d, z64, H)
    h2 = _epilogue(rst1, den1[0, :N], den1[1, :N], h1, b1.reshape(1, H * HID), act=True)

    Wc = jnp.concatenate([W2, res2], axis=1)
    ALc2 = jnp.zeros((2 * C, 8), jnp.float32).at[:C, 0].set(al2[0])
    ARc2 = jnp.zeros((2 * C, 8), jnp.float32).at[:C, 0].set(ar2[0])
    featc, el2, er2 = _matmul_layer(h2, Wc, ALc2, ARc2, 2)
    ex2, den2 = _attn_den(el2, _pad_er(er2), src2d, dst2d, z8)
    rst2 = _msg(featc[0], ex2, src2d, dst2d, z64, 1)
    logits = _epilogue2(rst2[0], rst2[1], den2[0, :N], den2[1, :N],
                        featc[1], b2.reshape(1, C))
    return logits[:N - 1]
